# Initial kernel scaffold; baseline (speedup 1.0000x reference)
import jax
import jax.numpy as jnp
from jax.experimental import pallas as pl


def kernel(x, edge_index, edge_attr, node_W, node_b, edge_W, edge_b, l0_W1, l0_b1, l0_W2, l0_b2, l1_W1, l1_b1, l1_W2, l1_b2, l2_W1, l2_b1, l2_W2, l2_b2, l3_W1, l3_b1, l3_W2, l3_b2, f_W1, f_b1, f_W2, f_b2):
    raise NotImplementedError("write your pallas kernel here")


# speedup vs baseline: 2.9638x; 2.9638x over previous
import functools

import jax
import jax.numpy as jnp
from jax import lax
from jax.experimental import pallas as pl
from jax.experimental.pallas import tpu as pltpu
from jax.experimental.pallas import tpu_sc as plsc

N = 10000
E = 320000
D_NODE = 128
D_EDGE = 16
H = 64

NC = 2
NS = 16
NW = NC * NS

CHUNK = 128
EPT_CHUNKS = 79
E_PAD = NW * EPT_CHUNKS * CHUNK
DUMP = 10000
N_T = 10112
ROWS_PER_TILE = N_T // NS
W = H + 16

_MESH = plsc.VectorSubcoreMesh(core_axis_name="c", subcore_axis_name="s",
                               num_cores=NC, num_subcores=NS)


def _hp0_body(x_ref, nw_ref, nb_ref, w1_ref, out_ref):
    h0 = jnp.dot(x_ref[...], nw_ref[...], preferred_element_type=jnp.float32)
    h0 = h0 + nb_ref[...]
    out_ref[...] = jnp.dot(h0, w1_ref[0:H, :], preferred_element_type=jnp.float32)


def _eb_body(ea_ref, ew_ref, eb2_ref, w10, b10, w11, b11, w12, b12, w13, b13,
             o0, o1, o2, o3):
    ea = ea_ref[...]
    for w1_ref, b1_ref, o_ref in ((w10, b10, o0), (w11, b11, o1),
                                  (w12, b12, o2), (w13, b13, o3)):
        w1b = w1_ref[H:2 * H, :]
        wf = jnp.dot(ew_ref[...], w1b, preferred_element_type=jnp.float32)
        bf = jnp.dot(eb2_ref[...], w1b, preferred_element_type=jnp.float32) + b1_ref[...]
        o_ref[...] = jnp.dot(ea, wf, preferred_element_type=jnp.float32) + bf


def _fold_body(p_ref, w2_ref, b2_ref, w1n_ref, out_ref):
    s = p_ref[0] + p_ref[1]
    h = jnp.dot(s[:, 0:H], w2_ref[...], preferred_element_type=jnp.float32)
    h = h + s[:, H:H + 1] * b2_ref[...]
    out_ref[...] = jnp.dot(h, w1n_ref[0:H, :], preferred_element_type=jnp.float32)


def _ffold_body(p_ref, w2_ref, b2_ref, fw1_ref, fb1_ref, u_ref, v_ref):
    s = p_ref[0] + p_ref[1]
    h = jnp.dot(s[:, 0:H], w2_ref[...], preferred_element_type=jnp.float32)
    h = h + s[:, H:H + 1] * b2_ref[...]
    u_ref[...] = jnp.dot(h, fw1_ref[0:H, :], preferred_element_type=jnp.float32)
    v_ref[...] = jnp.dot(h, fw1_ref[H:2 * H, :],
                         preferred_element_type=jnp.float32) + fb1_ref[...]


def _sc_layer_body(hp_hbm, src_hbm, dst_hbm, eb_hbm, zeros_hbm, out_hbm,
                   src_v, dst_v, gs_v, gd_v, eb_v, r_v, acc, sem1, sem2):
    c = lax.axis_index("c")
    s = lax.axis_index("s")
    wid = c * NS + s

    @pl.when(s == 0)
    def _init():
        pltpu.sync_copy(zeros_hbm, acc)

    ones0 = jnp.where(lax.iota(jnp.int32, 16) == 0, 1.0, 0.0)

    def _crow(i, _):
        r_v[i, pl.ds(H, 16)] = ones0
        return 0
    lax.fori_loop(0, CHUNK, _crow, 0)
    plsc.subcore_barrier()

    def _chunk(cid, _):
        base = (wid * EPT_CHUNKS + cid) * CHUNK
        pltpu.sync_copy(src_hbm.at[pl.ds(base, CHUNK)], src_v)
        pltpu.sync_copy(dst_hbm.at[pl.ds(base, CHUNK)], dst_v)
        cp1 = pltpu.async_copy(hp_hbm.at[src_v], gs_v, sem1)
        cp2 = pltpu.async_copy(hp_hbm.at[dst_v], gd_v, sem2)
        pltpu.sync_copy(eb_hbm.at[pl.ds(base, CHUNK), :], eb_v)
        cp1.wait()
        cp2.wait()

        def _erow(e, _):
            for dc in range(H // 16):
                sl = pl.ds(dc * 16, 16)
                r_v[e, sl] = jnp.maximum(gs_v[e, sl] + gd_v[e, sl] + eb_v[e, sl],
                                         0.0)
            return 0
        lax.fori_loop(0, CHUNK, _erow, 0)
        pltpu.sync_copy(r_v, acc.at[dst_v], add=True)
        return 0
    lax.fori_loop(0, EPT_CHUNKS, _chunk, 0)
    plsc.subcore_barrier()

    pltpu.sync_copy(acc.at[pl.ds(s * ROWS_PER_TILE, ROWS_PER_TILE), :],
                    out_hbm.at[c, pl.ds(s * ROWS_PER_TILE, ROWS_PER_TILE), :])


def _sc_final_body(u_hbm, v_hbm, src_hbm, dst_hbm, w2b_hbm, out_hbm,
                   src_v, dst_v, gu_v, gv_v, w2_v, o_v, sem1, sem2):
    c = lax.axis_index("c")
    s = lax.axis_index("s")
    wid = c * NS + s
    pltpu.sync_copy(w2b_hbm, w2_v)

    def _chunk(cid, _):
        base = (wid * EPT_CHUNKS + cid) * CHUNK
        pltpu.sync_copy(src_hbm.at[pl.ds(base, CHUNK)], src_v)
        pltpu.sync_copy(dst_hbm.at[pl.ds(base, CHUNK)], dst_v)
        cp1 = pltpu.async_copy(u_hbm.at[src_v], gu_v, sem1)
        cp2 = pltpu.async_copy(v_hbm.at[dst_v], gv_v, sem2)
        cp1.wait()
        cp2.wait()
        bias = w2_v[pl.ds(H, 16)][0]
        lane = lax.iota(jnp.int32, 16)
        wv = [w2_v[pl.ds(dc * 16, 16)] for dc in range(H // 16)]

        def _egroup(g, _):
            out_vec = jnp.zeros((16,), jnp.float32)
            for k in range(16):
                e = g * 16 + k
                t = jnp.zeros((16,), jnp.float32)
                for dc in range(H // 16):
                    sl = pl.ds(dc * 16, 16)
                    t = t + jnp.maximum(gu_v[e, sl] + gv_v[e, sl], 0.0) * wv[dc]
                out_vec = jnp.where(lane == k, jnp.sum(t) + bias, out_vec)
            o_v[pl.ds(g * 16, 16)] = out_vec
            return 0
        lax.fori_loop(0, CHUNK // 16, _egroup, 0)
        pltpu.sync_copy(o_v, out_hbm.at[pl.ds(base, CHUNK)])
        return 0
    lax.fori_loop(0, EPT_CHUNKS, _chunk, 0)


_SC_PARAMS = pltpu.CompilerParams(use_tc_tiling_on_sc=False,
                                  needs_layout_passes=False)

_sc_layer = functools.partial(
    pl.kernel, _sc_layer_body,
    out_type=jax.ShapeDtypeStruct((NC, N_T, W), jnp.float32),
    mesh=_MESH,
    compiler_params=_SC_PARAMS,
    scratch_types=[
        pltpu.VMEM((CHUNK,), jnp.int32),
        pltpu.VMEM((CHUNK,), jnp.int32),
        pltpu.VMEM((CHUNK, H), jnp.float32),
        pltpu.VMEM((CHUNK, H), jnp.float32),
        pltpu.VMEM((CHUNK, H), jnp.float32),
        pltpu.VMEM((CHUNK, W), jnp.float32),
        pltpu.VMEM_SHARED((N_T, W), jnp.float32),
        pltpu.SemaphoreType.DMA,
        pltpu.SemaphoreType.DMA,
    ],
)()

_sc_final = functools.partial(
    pl.kernel, _sc_final_body,
    out_type=jax.ShapeDtypeStruct((E_PAD,), jnp.float32),
    mesh=_MESH,
    compiler_params=_SC_PARAMS,
    scratch_types=[
        pltpu.VMEM((CHUNK,), jnp.int32),
        pltpu.VMEM((CHUNK,), jnp.int32),
        pltpu.VMEM((CHUNK, H), jnp.float32),
        pltpu.VMEM((CHUNK, H), jnp.float32),
        pltpu.VMEM((W,), jnp.float32),
        pltpu.VMEM((CHUNK,), jnp.float32),
        pltpu.SemaphoreType.DMA,
        pltpu.SemaphoreType.DMA,
    ],
)()


def kernel(x, edge_index, edge_attr, node_W, node_b, edge_W, edge_b,
           l0_W1, l0_b1, l0_W2, l0_b2,
           l1_W1, l1_b1, l1_W2, l1_b2,
           l2_W1, l2_b1, l2_W2, l2_b2,
           l3_W1, l3_b1, l3_W2, l3_b2,
           f_W1, f_b1, f_W2, f_b2):
    f32 = jnp.float32
    src = jnp.pad(edge_index[0].astype(jnp.int32), (0, E_PAD - E))
    dst = jnp.pad(edge_index[1].astype(jnp.int32), (0, E_PAD - E),
                  constant_values=DUMP)
    ea_pad = jnp.pad(edge_attr, ((0, E_PAD - E), (0, 0)))
    x_pad = jnp.pad(x, ((0, N_T - N), (0, 0)))
    nb2 = node_b.reshape(1, H)
    eb2 = edge_b.reshape(1, H)
    b1s = [b.reshape(1, H) for b in (l0_b1, l1_b1, l2_b1, l3_b1)]
    b2s = [b.reshape(1, H) for b in (l0_b2, l1_b2, l2_b2, l3_b2)]
    fb1_2 = f_b1.reshape(1, H)
    w2b = jnp.concatenate([f_W2[:, 0], f_b2, jnp.zeros((15,), f32)])

    def full(shape):
        return pl.BlockSpec(shape, lambda i: (0, 0))

    hp0 = pl.pallas_call(
        _hp0_body,
        out_shape=jax.ShapeDtypeStruct((N_T, H), f32),
    )(x_pad, node_W, nb2, l0_W1)

    eb_grid = E_PAD // 4096
    ebs = pl.pallas_call(
        _eb_body,
        grid=(eb_grid,),
        in_specs=[pl.BlockSpec((4096, D_EDGE), lambda i: (i, 0)),
                  full((D_EDGE, H)), full((1, H)),
                  full((2 * H, H)), full((1, H)), full((2 * H, H)), full((1, H)),
                  full((2 * H, H)), full((1, H)), full((2 * H, H)), full((1, H))],
        out_specs=[pl.BlockSpec((4096, H), lambda i: (i, 0))] * 4,
        out_shape=[jax.ShapeDtypeStruct((E_PAD, H), f32)] * 4,
    )(ea_pad, edge_W, eb2,
      l0_W1, b1s[0], l1_W1, b1s[1], l2_W1, b1s[2], l3_W1, b1s[3])

    w1n = [l1_W1, l2_W1, l3_W1]
    w2s = [l0_W2, l1_W2, l2_W2, l3_W2]
    zeros_acc = jnp.zeros((N_T, W), f32)
    hp = hp0
    p = None
    for l in range(4):
        p = _sc_layer(hp, src, dst, ebs[l], zeros_acc)
        if l < 3:
            hp = pl.pallas_call(
                _fold_body,
                out_shape=jax.ShapeDtypeStruct((N_T, H), f32),
            )(p, w2s[l], b2s[l], w1n[l])

    u, v = pl.pallas_call(
        _ffold_body,
        out_shape=[jax.ShapeDtypeStruct((N_T, H), f32)] * 2,
    )(p, l3_W2, b2s[3], f_W1, fb1_2)

    out = _sc_final(u, v, src, dst, w2b)
    return out[:E]


# speedup vs baseline: 4.0294x; 1.3595x over previous
import functools

import jax
import jax.numpy as jnp
from jax import lax
from jax.experimental import pallas as pl
from jax.experimental.pallas import tpu as pltpu
from jax.experimental.pallas import tpu_sc as plsc

N = 10000
E = 320000
D_NODE = 128
D_EDGE = 16
H = 64

NC = 2
NS = 16
NW = NC * NS

CHUNK = 128
EPT_CHUNKS = 80
EPT_PAIRS = EPT_CHUNKS // 2
E_PAD = NW * EPT_CHUNKS * CHUNK
DUMP = 10000
N_T = 10112
ROWS_PER_TILE = N_T // NS
W = H + 16

_MESH = plsc.VectorSubcoreMesh(core_axis_name="c", subcore_axis_name="s",
                               num_cores=NC, num_subcores=NS)


def _hp0_body(x_ref, nw_ref, nb_ref, w1_ref, out_ref):
    h0 = jnp.dot(x_ref[...], nw_ref[...], preferred_element_type=jnp.float32)
    h0 = h0 + nb_ref[...]
    out_ref[...] = jnp.dot(h0, w1_ref[0:H, :], preferred_element_type=jnp.float32)


def _eb_body(ea_ref, ew_ref, eb2_ref, w10, b10, w11, b11, w12, b12, w13, b13,
             o0, o1, o2, o3):
    ea = ea_ref[...]
    for w1_ref, b1_ref, o_ref in ((w10, b10, o0), (w11, b11, o1),
                                  (w12, b12, o2), (w13, b13, o3)):
        w1b = w1_ref[H:2 * H, :]
        wf = jnp.dot(ew_ref[...], w1b, preferred_element_type=jnp.float32)
        bf = jnp.dot(eb2_ref[...], w1b, preferred_element_type=jnp.float32) + b1_ref[...]
        o_ref[...] = jnp.dot(ea, wf, preferred_element_type=jnp.float32) + bf


def _fold_body(p_ref, w2_ref, b2_ref, w1n_ref, out_ref):
    s = p_ref[0] + p_ref[1]
    h = jnp.dot(s[:, 0:H], w2_ref[...], preferred_element_type=jnp.float32)
    h = h + s[:, H:H + 1] * b2_ref[...]
    out_ref[...] = jnp.dot(h, w1n_ref[0:H, :], preferred_element_type=jnp.float32)


def _ffold_body(p_ref, w2_ref, b2_ref, fw1_ref, fb1_ref, u_ref, v_ref):
    s = p_ref[0] + p_ref[1]
    h = jnp.dot(s[:, 0:H], w2_ref[...], preferred_element_type=jnp.float32)
    h = h + s[:, H:H + 1] * b2_ref[...]
    u_ref[...] = jnp.dot(h, fw1_ref[0:H, :], preferred_element_type=jnp.float32)
    v_ref[...] = jnp.dot(h, fw1_ref[H:2 * H, :],
                         preferred_element_type=jnp.float32) + fb1_ref[...]


def _sc_layer_body(hp_hbm, src_hbm, dst_hbm, eb_hbm, zeros_hbm, out_hbm,
                   src0, src1, dst0, dst1, dsc0, dsc1,
                   gs0, gs1, gd0, gd1, eb0, eb1, r0, r1,
                   six0, six1, sg0, sg1, ss0, ss1, acc):
    c = lax.axis_index("c")
    s = lax.axis_index("s")
    wid = c * NS + s
    tbase = wid * EPT_CHUNKS

    srcv = (src0, src1)
    dstv = (dst0, dst1)
    dscv = (dsc0, dsc1)
    gsv = (gs0, gs1)
    gdv = (gd0, gd1)
    ebv = (eb0, eb1)
    rv = (r0, r1)
    six = (six0, six1)
    sg = (sg0, sg1)
    ss = (ss0, ss1)

    @pl.when(s == 0)
    def _init():
        pltpu.sync_copy(zeros_hbm, acc)

    ones0 = jnp.where(lax.iota(jnp.int32, 16) == 0, 1.0, 0.0)

    def _crow(i, _):
        r0[i, pl.ds(H, 16)] = ones0
        r1[i, pl.ds(H, 16)] = ones0
        return 0
    lax.fori_loop(0, CHUNK, _crow, 0)
    plsc.subcore_barrier()

    def issue_idx(cid, b):
        base = (tbase + cid) * CHUNK
        pltpu.async_copy(src_hbm.at[pl.ds(base, CHUNK)], srcv[b], six[b])
        pltpu.async_copy(dst_hbm.at[pl.ds(base, CHUNK)], dstv[b], six[b])

    def wait_idx(b):
        pltpu.make_async_copy(src_hbm.at[pl.ds(0, CHUNK)], srcv[b], six[b]).wait()
        pltpu.make_async_copy(dst_hbm.at[pl.ds(0, CHUNK)], dstv[b], six[b]).wait()

    def issue_gather(cid, b):
        base = (tbase + cid) * CHUNK
        pltpu.async_copy(hp_hbm.at[srcv[b]], gsv[b], sg[b])
        pltpu.async_copy(hp_hbm.at[dstv[b]], gdv[b], sg[b])
        pltpu.async_copy(eb_hbm.at[pl.ds(base, CHUNK), :], ebv[b], sg[b])

    def wait_gather(b):
        pltpu.make_async_copy(hp_hbm.at[srcv[b]], gsv[b], sg[b]).wait()
        pltpu.make_async_copy(hp_hbm.at[dstv[b]], gdv[b], sg[b]).wait()
        pltpu.make_async_copy(eb_hbm.at[pl.ds(0, CHUNK), :], ebv[b], sg[b]).wait()

    def compute(b):
        gs_b, gd_b, eb_b, r_b = gsv[b], gdv[b], ebv[b], rv[b]

        def _erow(e, _):
            for dc in range(H // 16):
                sl = pl.ds(dc * 16, 16)
                r_b[e, sl] = jnp.maximum(gs_b[e, sl] + gd_b[e, sl] + eb_b[e, sl],
                                         0.0)
            return 0
        lax.fori_loop(0, CHUNK, _erow, 0, unroll=4)

    def issue_scatter(b):
        for k in range(CHUNK // 16):
            sl = pl.ds(k * 16, 16)
            dscv[b][sl] = dstv[b][sl]
        pltpu.async_copy(rv[b], acc.at[dscv[b]], ss[b], add=True)

    def wait_scatter(b):
        pltpu.make_async_copy(rv[b], acc.at[dscv[b]], ss[b]).wait()

    issue_idx(0, 0)
    issue_idx(1, 1)
    wait_idx(0)
    issue_gather(0, 0)

    def _pair(ii, _):
        c0 = ii * 2
        wait_gather(0)
        wait_idx(1)
        issue_gather(c0 + 1, 1)

        @pl.when(ii >= 1)
        def _():
            wait_scatter(0)
        compute(0)
        issue_scatter(0)

        @pl.when(ii <= EPT_PAIRS - 2)
        def _():
            issue_idx(c0 + 2, 0)

        wait_gather(1)

        @pl.when(ii <= EPT_PAIRS - 2)
        def _():
            wait_idx(0)
            issue_gather(c0 + 2, 0)

        @pl.when(ii >= 1)
        def _():
            wait_scatter(1)
        compute(1)
        issue_scatter(1)

        @pl.when(ii <= EPT_PAIRS - 2)
        def _():
            issue_idx(c0 + 3, 1)
        return 0
    lax.fori_loop(0, EPT_PAIRS, _pair, 0)
    wait_scatter(0)
    wait_scatter(1)
    plsc.subcore_barrier()

    pltpu.sync_copy(acc.at[pl.ds(s * ROWS_PER_TILE, ROWS_PER_TILE), :],
                    out_hbm.at[c, pl.ds(s * ROWS_PER_TILE, ROWS_PER_TILE), :])


def _sc_final_body(u_hbm, v_hbm, src_hbm, dst_hbm, w2b_hbm, out_hbm,
                   src0, src1, dst0, dst1, gu0, gu1, gv0, gv1, o0, o1,
                   w2_v, six0, six1, sg0, sg1, so0, so1):
    c = lax.axis_index("c")
    s = lax.axis_index("s")
    wid = c * NS + s
    tbase = wid * EPT_CHUNKS
    pltpu.sync_copy(w2b_hbm, w2_v)

    srcv = (src0, src1)
    dstv = (dst0, dst1)
    guv = (gu0, gu1)
    gvv = (gv0, gv1)
    ov = (o0, o1)
    six = (six0, six1)
    sg = (sg0, sg1)
    so = (so0, so1)

    bias = w2_v[pl.ds(H, 16)][0]
    lane = lax.iota(jnp.int32, 16)
    wv = [w2_v[pl.ds(dc * 16, 16)] for dc in range(H // 16)]

    def issue_idx(cid, b):
        base = (tbase + cid) * CHUNK
        pltpu.async_copy(src_hbm.at[pl.ds(base, CHUNK)], srcv[b], six[b])
        pltpu.async_copy(dst_hbm.at[pl.ds(base, CHUNK)], dstv[b], six[b])

    def wait_idx(b):
        pltpu.make_async_copy(src_hbm.at[pl.ds(0, CHUNK)], srcv[b], six[b]).wait()
        pltpu.make_async_copy(dst_hbm.at[pl.ds(0, CHUNK)], dstv[b], six[b]).wait()

    def issue_gather(b):
        pltpu.async_copy(u_hbm.at[srcv[b]], guv[b], sg[b])
        pltpu.async_copy(v_hbm.at[dstv[b]], gvv[b], sg[b])

    def wait_gather(b):
        pltpu.make_async_copy(u_hbm.at[srcv[b]], guv[b], sg[b]).wait()
        pltpu.make_async_copy(v_hbm.at[dstv[b]], gvv[b], sg[b]).wait()

    def compute(b):
        gu_b, gv_b, o_b = guv[b], gvv[b], ov[b]

        def _egroup(g, _):
            out_vec = jnp.zeros((16,), jnp.float32)
            for k in range(16):
                e = g * 16 + k
                t = jnp.zeros((16,), jnp.float32)
                for dc in range(H // 16):
                    sl = pl.ds(dc * 16, 16)
                    t = t + jnp.maximum(gu_b[e, sl] + gv_b[e, sl], 0.0) * wv[dc]
                out_vec = jnp.where(lane == k, jnp.sum(t) + bias, out_vec)
            o_b[pl.ds(g * 16, 16)] = out_vec
            return 0
        lax.fori_loop(0, CHUNK // 16, _egroup, 0)

    def issue_store(cid, b):
        base = (tbase + cid) * CHUNK
        pltpu.async_copy(ov[b], out_hbm.at[pl.ds(base, CHUNK)], so[b])

    def wait_store(b):
        pltpu.make_async_copy(ov[b], out_hbm.at[pl.ds(0, CHUNK)], so[b]).wait()

    issue_idx(0, 0)
    issue_idx(1, 1)
    wait_idx(0)
    issue_gather(0)

    def _pair(ii, _):
        c0 = ii * 2
        wait_gather(0)
        wait_idx(1)
        issue_gather(1)

        @pl.when(ii >= 1)
        def _():
            wait_store(0)
        compute(0)
        issue_store(c0, 0)

        @pl.when(ii <= EPT_PAIRS - 2)
        def _():
            issue_idx(c0 + 2, 0)

        wait_gather(1)

        @pl.when(ii <= EPT_PAIRS - 2)
        def _():
            wait_idx(0)
            issue_gather(0)

        @pl.when(ii >= 1)
        def _():
            wait_store(1)
        compute(1)
        issue_store(c0 + 1, 1)

        @pl.when(ii <= EPT_PAIRS - 2)
        def _():
            issue_idx(c0 + 3, 1)
        return 0
    lax.fori_loop(0, EPT_PAIRS, _pair, 0)
    wait_store(0)
    wait_store(1)


_SC_PARAMS = pltpu.CompilerParams(use_tc_tiling_on_sc=False,
                                  needs_layout_passes=False)

_sc_layer = functools.partial(
    pl.kernel, _sc_layer_body,
    out_type=jax.ShapeDtypeStruct((NC, N_T, W), jnp.float32),
    mesh=_MESH,
    compiler_params=_SC_PARAMS,
    scratch_types=(
        [pltpu.VMEM((CHUNK,), jnp.int32)] * 6
        + [pltpu.VMEM((CHUNK, H), jnp.float32)] * 6
        + [pltpu.VMEM((CHUNK, W), jnp.float32)] * 2
        + [pltpu.SemaphoreType.DMA] * 6
        + [pltpu.VMEM_SHARED((N_T, W), jnp.float32)]
    ),
)()

_sc_final = functools.partial(
    pl.kernel, _sc_final_body,
    out_type=jax.ShapeDtypeStruct((E_PAD,), jnp.float32),
    mesh=_MESH,
    compiler_params=_SC_PARAMS,
    scratch_types=(
        [pltpu.VMEM((CHUNK,), jnp.int32)] * 4
        + [pltpu.VMEM((CHUNK, H), jnp.float32)] * 4
        + [pltpu.VMEM((CHUNK,), jnp.float32)] * 2
        + [pltpu.VMEM((W,), jnp.float32)]
        + [pltpu.SemaphoreType.DMA] * 6
    ),
)()


def kernel(x, edge_index, edge_attr, node_W, node_b, edge_W, edge_b,
           l0_W1, l0_b1, l0_W2, l0_b2,
           l1_W1, l1_b1, l1_W2, l1_b2,
           l2_W1, l2_b1, l2_W2, l2_b2,
           l3_W1, l3_b1, l3_W2, l3_b2,
           f_W1, f_b1, f_W2, f_b2):
    f32 = jnp.float32
    src = jnp.pad(edge_index[0].astype(jnp.int32), (0, E_PAD - E))
    dst = jnp.pad(edge_index[1].astype(jnp.int32), (0, E_PAD - E),
                  constant_values=DUMP)
    ea_pad = jnp.pad(edge_attr, ((0, E_PAD - E), (0, 0)))
    x_pad = jnp.pad(x, ((0, N_T - N), (0, 0)))
    nb2 = node_b.reshape(1, H)
    eb2 = edge_b.reshape(1, H)
    b1s = [b.reshape(1, H) for b in (l0_b1, l1_b1, l2_b1, l3_b1)]
    b2s = [b.reshape(1, H) for b in (l0_b2, l1_b2, l2_b2, l3_b2)]
    fb1_2 = f_b1.reshape(1, H)
    w2b = jnp.concatenate([f_W2[:, 0], f_b2, jnp.zeros((15,), f32)])

    def full(shape):
        return pl.BlockSpec(shape, lambda i: (0, 0))

    hp0 = pl.pallas_call(
        _hp0_body,
        out_shape=jax.ShapeDtypeStruct((N_T, H), f32),
    )(x_pad, node_W, nb2, l0_W1)

    eb_grid = E_PAD // 4096
    ebs = pl.pallas_call(
        _eb_body,
        grid=(eb_grid,),
        in_specs=[pl.BlockSpec((4096, D_EDGE), lambda i: (i, 0)),
                  full((D_EDGE, H)), full((1, H)),
                  full((2 * H, H)), full((1, H)), full((2 * H, H)), full((1, H)),
                  full((2 * H, H)), full((1, H)), full((2 * H, H)), full((1, H))],
        out_specs=[pl.BlockSpec((4096, H), lambda i: (i, 0))] * 4,
        out_shape=[jax.ShapeDtypeStruct((E_PAD, H), f32)] * 4,
    )(ea_pad, edge_W, eb2,
      l0_W1, b1s[0], l1_W1, b1s[1], l2_W1, b1s[2], l3_W1, b1s[3])

    w1n = [l1_W1, l2_W1, l3_W1]
    w2s = [l0_W2, l1_W2, l2_W2, l3_W2]
    zeros_acc = jnp.zeros((N_T, W), f32)
    hp = hp0
    p = None
    for l in range(4):
        p = _sc_layer(hp, src, dst, ebs[l], zeros_acc)
        if l < 3:
            hp = pl.pallas_call(
                _fold_body,
                out_shape=jax.ShapeDtypeStruct((N_T, H), f32),
            )(p, w2s[l], b2s[l], w1n[l])

    u, v = pl.pallas_call(
        _ffold_body,
        out_shape=[jax.ShapeDtypeStruct((N_T, H), f32)] * 2,
    )(p, l3_W2, b2s[3], f_W1, fb1_2)

    out = _sc_final(u, v, src, dst, w2b)
    return out[:E]


# speedup vs baseline: 4.0299x; 1.0001x over previous
import functools

import jax
import jax.numpy as jnp
from jax import lax
from jax.experimental import pallas as pl
from jax.experimental.pallas import tpu as pltpu
from jax.experimental.pallas import tpu_sc as plsc

N = 10000
E = 320000
D_NODE = 128
D_EDGE = 16
H = 64

NC = 2
NS = 16
NW = NC * NS

CHUNK = 128
EPT_CHUNKS = 80
EPT_PAIRS = EPT_CHUNKS // 2
E_PAD = NW * EPT_CHUNKS * CHUNK
DUMP = 10000
N_T = 10112
ROWS_PER_TILE = N_T // NS
W = H + 16

_MESH = plsc.VectorSubcoreMesh(core_axis_name="c", subcore_axis_name="s",
                               num_cores=NC, num_subcores=NS)


def _hp0_body(x_ref, nw_ref, nb_ref, w1_ref, out_ref):
    h0 = jnp.dot(x_ref[...], nw_ref[...], preferred_element_type=jnp.float32)
    h0 = h0 + nb_ref[...]
    out_ref[...] = jnp.dot(h0, w1_ref[0:H, :], preferred_element_type=jnp.float32)


def _eb_body(ea_ref, ew_ref, eb2_ref, w10, b10, w11, b11, w12, b12, w13, b13,
             o0, o1, o2, o3):
    ea = ea_ref[...]
    for w1_ref, b1_ref, o_ref in ((w10, b10, o0), (w11, b11, o1),
                                  (w12, b12, o2), (w13, b13, o3)):
        w1b = w1_ref[H:2 * H, :]
        wf = jnp.dot(ew_ref[...], w1b, preferred_element_type=jnp.float32)
        bf = jnp.dot(eb2_ref[...], w1b, preferred_element_type=jnp.float32) + b1_ref[...]
        o_ref[...] = jnp.dot(ea, wf, preferred_element_type=jnp.float32) + bf


def _fold_body(p_ref, w2_ref, b2_ref, w1n_ref, out_ref):
    s = p_ref[0] + p_ref[1]
    h = jnp.dot(s[:, 0:H], w2_ref[...], preferred_element_type=jnp.float32)
    h = h + s[:, H:H + 1] * b2_ref[...]
    out_ref[...] = jnp.dot(h, w1n_ref[0:H, :], preferred_element_type=jnp.float32)


def _ffold_body(p_ref, w2_ref, b2_ref, fw1_ref, fb1_ref, u_ref, v_ref):
    s = p_ref[0] + p_ref[1]
    h = jnp.dot(s[:, 0:H], w2_ref[...], preferred_element_type=jnp.float32)
    h = h + s[:, H:H + 1] * b2_ref[...]
    u_ref[...] = jnp.dot(h, fw1_ref[0:H, :], preferred_element_type=jnp.float32)
    v_ref[...] = jnp.dot(h, fw1_ref[H:2 * H, :],
                         preferred_element_type=jnp.float32) + fb1_ref[...]


def _sc_layer_body(hp_hbm, src_hbm, dst_hbm, eb_hbm, zeros_hbm, out_hbm,
                   src0, src1, dst0, dst1, dsc0, dsc1,
                   gs0, gs1, gd0, gd1, eb0, eb1, r0, r1,
                   six0, six1, sg0, sg1, ss0, ss1, acc):
    c = lax.axis_index("c")
    s = lax.axis_index("s")
    wid = c * NS + s
    tbase = wid * EPT_CHUNKS

    srcv = (src0, src1)
    dstv = (dst0, dst1)
    dscv = (dsc0, dsc1)
    gsv = (gs0, gs1)
    gdv = (gd0, gd1)
    ebv = (eb0, eb1)
    rv = (r0, r1)
    six = (six0, six1)
    sg = (sg0, sg1)
    ss = (ss0, ss1)

    @pl.when(s == 0)
    def _init():
        pltpu.sync_copy(zeros_hbm, acc)

    ones0 = jnp.where(lax.iota(jnp.int32, 16) == 0, 1.0, 0.0)

    def _crow(i, _):
        r0[i, pl.ds(H, 16)] = ones0
        r1[i, pl.ds(H, 16)] = ones0
        return 0
    lax.fori_loop(0, CHUNK, _crow, 0)
    plsc.subcore_barrier()

    def issue_idx(cid, b):
        base = (tbase + cid) * CHUNK
        pltpu.async_copy(src_hbm.at[pl.ds(base, CHUNK)], srcv[b], six[b])
        pltpu.async_copy(dst_hbm.at[pl.ds(base, CHUNK)], dstv[b], six[b])

    def wait_idx(b):
        pltpu.make_async_copy(src_hbm.at[pl.ds(0, CHUNK)], srcv[b], six[b]).wait()
        pltpu.make_async_copy(dst_hbm.at[pl.ds(0, CHUNK)], dstv[b], six[b]).wait()

    def issue_gather(cid, b):
        base = (tbase + cid) * CHUNK
        pltpu.async_copy(hp_hbm.at[srcv[b]], gsv[b], sg[b])
        pltpu.async_copy(hp_hbm.at[dstv[b]], gdv[b], sg[b])
        pltpu.async_copy(eb_hbm.at[pl.ds(base, CHUNK), :], ebv[b], sg[b])

    def wait_gather(b):
        pltpu.make_async_copy(hp_hbm.at[srcv[b]], gsv[b], sg[b]).wait()
        pltpu.make_async_copy(hp_hbm.at[dstv[b]], gdv[b], sg[b]).wait()
        pltpu.make_async_copy(eb_hbm.at[pl.ds(0, CHUNK), :], ebv[b], sg[b]).wait()

    def compute(b):
        gs_b, gd_b, eb_b, r_b = gsv[b], gdv[b], ebv[b], rv[b]

        def _erow(e, _):
            for dc in range(H // 16):
                sl = pl.ds(dc * 16, 16)
                r_b[e, sl] = jnp.maximum(gs_b[e, sl] + gd_b[e, sl] + eb_b[e, sl],
                                         0.0)
            return 0
        lax.fori_loop(0, CHUNK, _erow, 0, unroll=8)

    def issue_scatter(b):
        for k in range(CHUNK // 16):
            sl = pl.ds(k * 16, 16)
            dscv[b][sl] = dstv[b][sl]
        pltpu.async_copy(rv[b], acc.at[dscv[b]], ss[b], add=True)

    def wait_scatter(b):
        pltpu.make_async_copy(rv[b], acc.at[dscv[b]], ss[b]).wait()

    issue_idx(0, 0)
    issue_idx(1, 1)
    wait_idx(0)
    issue_gather(0, 0)

    def _pair(ii, _):
        c0 = ii * 2
        wait_gather(0)
        wait_idx(1)
        issue_gather(c0 + 1, 1)

        @pl.when(ii >= 1)
        def _():
            wait_scatter(0)
        compute(0)
        issue_scatter(0)

        @pl.when(ii <= EPT_PAIRS - 2)
        def _():
            issue_idx(c0 + 2, 0)

        wait_gather(1)

        @pl.when(ii <= EPT_PAIRS - 2)
        def _():
            wait_idx(0)
            issue_gather(c0 + 2, 0)

        @pl.when(ii >= 1)
        def _():
            wait_scatter(1)
        compute(1)
        issue_scatter(1)

        @pl.when(ii <= EPT_PAIRS - 2)
        def _():
            issue_idx(c0 + 3, 1)
        return 0
    lax.fori_loop(0, EPT_PAIRS, _pair, 0)
    wait_scatter(0)
    wait_scatter(1)
    plsc.subcore_barrier()

    pltpu.sync_copy(acc.at[pl.ds(s * ROWS_PER_TILE, ROWS_PER_TILE), :],
                    out_hbm.at[c, pl.ds(s * ROWS_PER_TILE, ROWS_PER_TILE), :])


def _sc_final_body(u_hbm, v_hbm, src_hbm, dst_hbm, w2b_hbm, out_hbm,
                   src0, src1, dst0, dst1, gu0, gu1, gv0, gv1, o0, o1,
                   w2_v, six0, six1, sg0, sg1, so0, so1):
    c = lax.axis_index("c")
    s = lax.axis_index("s")
    wid = c * NS + s
    tbase = wid * EPT_CHUNKS
    pltpu.sync_copy(w2b_hbm, w2_v)

    srcv = (src0, src1)
    dstv = (dst0, dst1)
    guv = (gu0, gu1)
    gvv = (gv0, gv1)
    ov = (o0, o1)
    six = (six0, six1)
    sg = (sg0, sg1)
    so = (so0, so1)

    bias = w2_v[pl.ds(H, 16)][0]
    lane = lax.iota(jnp.int32, 16)
    wv = [w2_v[pl.ds(dc * 16, 16)] for dc in range(H // 16)]

    def issue_idx(cid, b):
        base = (tbase + cid) * CHUNK
        pltpu.async_copy(src_hbm.at[pl.ds(base, CHUNK)], srcv[b], six[b])
        pltpu.async_copy(dst_hbm.at[pl.ds(base, CHUNK)], dstv[b], six[b])

    def wait_idx(b):
        pltpu.make_async_copy(src_hbm.at[pl.ds(0, CHUNK)], srcv[b], six[b]).wait()
        pltpu.make_async_copy(dst_hbm.at[pl.ds(0, CHUNK)], dstv[b], six[b]).wait()

    def issue_gather(b):
        pltpu.async_copy(u_hbm.at[srcv[b]], guv[b], sg[b])
        pltpu.async_copy(v_hbm.at[dstv[b]], gvv[b], sg[b])

    def wait_gather(b):
        pltpu.make_async_copy(u_hbm.at[srcv[b]], guv[b], sg[b]).wait()
        pltpu.make_async_copy(v_hbm.at[dstv[b]], gvv[b], sg[b]).wait()

    def compute(b):
        gu_b, gv_b, o_b = guv[b], gvv[b], ov[b]

        def _egroup(g, _):
            out_vec = jnp.zeros((16,), jnp.float32)
            for k in range(16):
                e = g * 16 + k
                t = jnp.zeros((16,), jnp.float32)
                for dc in range(H // 16):
                    sl = pl.ds(dc * 16, 16)
                    t = t + jnp.maximum(gu_b[e, sl] + gv_b[e, sl], 0.0) * wv[dc]
                out_vec = jnp.where(lane == k, jnp.sum(t) + bias, out_vec)
            o_b[pl.ds(g * 16, 16)] = out_vec
            return 0
        lax.fori_loop(0, CHUNK // 16, _egroup, 0)

    def issue_store(cid, b):
        base = (tbase + cid) * CHUNK
        pltpu.async_copy(ov[b], out_hbm.at[pl.ds(base, CHUNK)], so[b])

    def wait_store(b):
        pltpu.make_async_copy(ov[b], out_hbm.at[pl.ds(0, CHUNK)], so[b]).wait()

    issue_idx(0, 0)
    issue_idx(1, 1)
    wait_idx(0)
    issue_gather(0)

    def _pair(ii, _):
        c0 = ii * 2
        wait_gather(0)
        wait_idx(1)
        issue_gather(1)

        @pl.when(ii >= 1)
        def _():
            wait_store(0)
        compute(0)
        issue_store(c0, 0)

        @pl.when(ii <= EPT_PAIRS - 2)
        def _():
            issue_idx(c0 + 2, 0)

        wait_gather(1)

        @pl.when(ii <= EPT_PAIRS - 2)
        def _():
            wait_idx(0)
            issue_gather(0)

        @pl.when(ii >= 1)
        def _():
            wait_store(1)
        compute(1)
        issue_store(c0 + 1, 1)

        @pl.when(ii <= EPT_PAIRS - 2)
        def _():
            issue_idx(c0 + 3, 1)
        return 0
    lax.fori_loop(0, EPT_PAIRS, _pair, 0)
    wait_store(0)
    wait_store(1)


_SC_PARAMS = pltpu.CompilerParams(use_tc_tiling_on_sc=False,
                                  needs_layout_passes=False)

_sc_layer = functools.partial(
    pl.kernel, _sc_layer_body,
    out_type=jax.ShapeDtypeStruct((NC, N_T, W), jnp.float32),
    mesh=_MESH,
    compiler_params=_SC_PARAMS,
    scratch_types=(
        [pltpu.VMEM((CHUNK,), jnp.int32)] * 6
        + [pltpu.VMEM((CHUNK, H), jnp.float32)] * 6
        + [pltpu.VMEM((CHUNK, W), jnp.float32)] * 2
        + [pltpu.SemaphoreType.DMA] * 6
        + [pltpu.VMEM_SHARED((N_T, W), jnp.float32)]
    ),
)()

_sc_final = functools.partial(
    pl.kernel, _sc_final_body,
    out_type=jax.ShapeDtypeStruct((E_PAD,), jnp.float32),
    mesh=_MESH,
    compiler_params=_SC_PARAMS,
    scratch_types=(
        [pltpu.VMEM((CHUNK,), jnp.int32)] * 4
        + [pltpu.VMEM((CHUNK, H), jnp.float32)] * 4
        + [pltpu.VMEM((CHUNK,), jnp.float32)] * 2
        + [pltpu.VMEM((W,), jnp.float32)]
        + [pltpu.SemaphoreType.DMA] * 6
    ),
)()


def kernel(x, edge_index, edge_attr, node_W, node_b, edge_W, edge_b,
           l0_W1, l0_b1, l0_W2, l0_b2,
           l1_W1, l1_b1, l1_W2, l1_b2,
           l2_W1, l2_b1, l2_W2, l2_b2,
           l3_W1, l3_b1, l3_W2, l3_b2,
           f_W1, f_b1, f_W2, f_b2):
    f32 = jnp.float32
    src = jnp.pad(edge_index[0].astype(jnp.int32), (0, E_PAD - E))
    dst = jnp.pad(edge_index[1].astype(jnp.int32), (0, E_PAD - E),
                  constant_values=DUMP)
    ea_pad = jnp.pad(edge_attr, ((0, E_PAD - E), (0, 0)))
    x_pad = jnp.pad(x, ((0, N_T - N), (0, 0)))
    nb2 = node_b.reshape(1, H)
    eb2 = edge_b.reshape(1, H)
    b1s = [b.reshape(1, H) for b in (l0_b1, l1_b1, l2_b1, l3_b1)]
    b2s = [b.reshape(1, H) for b in (l0_b2, l1_b2, l2_b2, l3_b2)]
    fb1_2 = f_b1.reshape(1, H)
    w2b = jnp.concatenate([f_W2[:, 0], f_b2, jnp.zeros((15,), f32)])

    def full(shape):
        return pl.BlockSpec(shape, lambda i: (0, 0))

    hp0 = pl.pallas_call(
        _hp0_body,
        out_shape=jax.ShapeDtypeStruct((N_T, H), f32),
    )(x_pad, node_W, nb2, l0_W1)

    eb_grid = E_PAD // 4096
    ebs = pl.pallas_call(
        _eb_body,
        grid=(eb_grid,),
        in_specs=[pl.BlockSpec((4096, D_EDGE), lambda i: (i, 0)),
                  full((D_EDGE, H)), full((1, H)),
                  full((2 * H, H)), full((1, H)), full((2 * H, H)), full((1, H)),
                  full((2 * H, H)), full((1, H)), full((2 * H, H)), full((1, H))],
        out_specs=[pl.BlockSpec((4096, H), lambda i: (i, 0))] * 4,
        out_shape=[jax.ShapeDtypeStruct((E_PAD, H), f32)] * 4,
    )(ea_pad, edge_W, eb2,
      l0_W1, b1s[0], l1_W1, b1s[1], l2_W1, b1s[2], l3_W1, b1s[3])

    w1n = [l1_W1, l2_W1, l3_W1]
    w2s = [l0_W2, l1_W2, l2_W2, l3_W2]
    zeros_acc = jnp.zeros((N_T, W), f32)
    hp = hp0
    p = None
    for l in range(4):
        p = _sc_layer(hp, src, dst, ebs[l], zeros_acc)
        if l < 3:
            hp = pl.pallas_call(
                _fold_body,
                out_shape=jax.ShapeDtypeStruct((N_T, H), f32),
            )(p, w2s[l], b2s[l], w1n[l])

    u, v = pl.pallas_call(
        _ffold_body,
        out_shape=[jax.ShapeDtypeStruct((N_T, H), f32)] * 2,
    )(p, l3_W2, b2s[3], f_W1, fb1_2)

    out = _sc_final(u, v, src, dst, w2b)
    return out[:E]
